# Initial kernel scaffold; baseline (speedup 1.0000x reference)
import jax
import jax.numpy as jnp
from jax.experimental import pallas as pl


def kernel(edge_index_list, node_feats_list, edge_feats_list, nodes_mask_list, W_i, W_f, W_c, W_o, b_i, b_f, b_c, b_o, conv_i_W, conv_i_b, conv_f_W, conv_f_b, conv_c_W, conv_c_b, conv_o_W, conv_o_b):
    raise NotImplementedError("write your pallas kernel here")


# speedup vs baseline: 14.8783x; 14.8783x over previous
import functools

import jax
import jax.numpy as jnp
from jax import lax
from jax.experimental import pallas as pl
from jax.experimental.pallas import tpu as pltpu
from jax.experimental.pallas import tpu_sc as plsc

NC = 2
NS = 16
NW = NC * NS


def _sc_mesh():
    return plsc.VectorSubcoreMesh(
        core_axis_name="c", subcore_axis_name="s",
        num_cores=NC, num_subcores=NS)


def _make_deg(E, N, RB):
    EPT = E // NW
    B = 400
    assert EPT % B == 0 and B % 16 == 0 and N % 16 == 0 and N % RB == 0

    @functools.partial(
        pl.kernel, mesh=_sc_mesh(),
        out_type=jax.ShapeDtypeStruct((N // RB, NW, RB), jnp.float32),
        compiler_params=pltpu.CompilerParams(
            needs_layout_passes=False, use_tc_tiling_on_sc=False),
        scratch_types=[
            pltpu.VMEM((N,), jnp.float32),
            pltpu.VMEM((B,), jnp.int32),
            pltpu.VMEM((B,), jnp.float32),
        ],
    )
    def deg_kernel(src_hbm, ew_hbm, out_hbm, acc_v, sidx_v, ew_v):
        cid = lax.axis_index("c")
        sid = lax.axis_index("s")
        wid = sid * NC + cid
        zero = jnp.zeros((16,), jnp.float32)

        def zbody(i, carry):
            acc_v[pl.ds(i * 16, 16)] = zero
            return carry
        lax.fori_loop(0, N // 16, zbody, 0)

        base = wid * EPT

        def blk(b, carry):
            off = base + b * B
            pltpu.sync_copy(src_hbm.at[pl.ds(off, B)], sidx_v)
            pltpu.sync_copy(ew_hbm.at[pl.ds(off, B)], ew_v)

            def grp(g, c2):
                idx = sidx_v[pl.ds(g * 16, 16)]
                w = ew_v[pl.ds(g * 16, 16)]
                plsc.addupdate_scatter(acc_v, [idx], w)
                return c2
            lax.fori_loop(0, B // 16, grp, 0)
            return carry
        lax.fori_loop(0, EPT // B, blk, 0)

        for q in range(N // RB):
            pltpu.sync_copy(acc_v.at[pl.ds(q * RB, RB)], out_hbm.at[q, wid])

    return deg_kernel


def _make_prop(E, N, D):
    EPT = E // NW
    B = 80
    RPS = N // NS
    ZR = 125
    assert EPT % B == 0 and B % 16 == 0 and RPS % ZR == 0 and D % 16 == 0

    @functools.partial(
        pl.kernel, mesh=_sc_mesh(),
        out_type=jax.ShapeDtypeStruct((NC, N, D), jnp.float32),
        compiler_params=pltpu.CompilerParams(
            needs_layout_passes=False, use_tc_tiling_on_sc=False),
        scratch_types=[
            pltpu.VMEM_SHARED((N, D), jnp.float32),
            pltpu.VMEM((B, D), jnp.float32),
            pltpu.VMEM((B,), jnp.int32),
            pltpu.VMEM((B,), jnp.int32),
            pltpu.VMEM((B,), jnp.float32),
            pltpu.VMEM((ZR, D), jnp.float32),
            pltpu.SemaphoreType.DMA,
        ],
    )
    def prop_kernel(u_hbm, src_hbm, dst_hbm, ew_hbm, out_hbm,
                    acc_sh, rows_v, sidx_v, didx_v, ew_v, zbuf, sem):
        cid = lax.axis_index("c")
        sid = lax.axis_index("s")
        wid = sid * NC + cid
        zero = jnp.zeros((16,), jnp.float32)

        def zrow(i, carry):
            for c in range(D // 16):
                zbuf[i, pl.ds(c * 16, 16)] = zero
            return carry
        lax.fori_loop(0, ZR, zrow, 0)
        for q in range(RPS // ZR):
            pltpu.sync_copy(zbuf, acc_sh.at[pl.ds(sid * RPS + q * ZR, ZR)])
        plsc.subcore_barrier()

        base = wid * EPT

        def blk(b, carry):
            off = base + b * B
            pltpu.sync_copy(src_hbm.at[pl.ds(off, B)], sidx_v)
            pltpu.sync_copy(dst_hbm.at[pl.ds(off, B)], didx_v)
            pltpu.sync_copy(ew_hbm.at[pl.ds(off, B)], ew_v)
            pltpu.async_copy(u_hbm.at[sidx_v], rows_v, sem).wait()

            def scale(g, c2):
                w16 = ew_v[pl.ds(g * 16, 16)]
                for j in range(16):
                    w = w16[j]
                    row = g * 16 + j
                    for c in range(D // 16):
                        rows_v[row, pl.ds(c * 16, 16)] = (
                            rows_v[row, pl.ds(c * 16, 16)] * w)
                return c2
            lax.fori_loop(0, B // 16, scale, 0)

            pltpu.sync_copy(rows_v, acc_sh.at[didx_v], add=True)
            return carry
        lax.fori_loop(0, EPT // B, blk, 0)

        plsc.subcore_barrier()
        pltpu.sync_copy(acc_sh.at[pl.ds(sid * RPS, RPS)],
                        out_hbm.at[cid, pl.ds(sid * RPS, RPS)])

    return prop_kernel


def _gates(z, D):
    zi = jax.nn.sigmoid(z[:, :D])
    zf = jax.nn.sigmoid(z[:, D:2 * D])
    zc = jnp.tanh(z[:, 2 * D:3 * D])
    zo = jax.nn.sigmoid(z[:, 3 * D:])
    return zi, zf, zc, zo


def _make_dense0(N, D, RB):
    def body(x_ref, w_ref, b_ref, h1_ref, c1_ref):
        z = jnp.dot(x_ref[...], w_ref[...],
                    preferred_element_type=jnp.float32) + b_ref[...]
        zi, zf, zc, zo = _gates(z, D)
        c1 = zi * zc
        c1_ref[...] = c1
        h1_ref[...] = zo * jnp.tanh(c1)

    return pl.pallas_call(
        body,
        grid=(N // RB,),
        in_specs=[
            pl.BlockSpec((RB, D), lambda i: (i, 0)),
            pl.BlockSpec((D, 4 * D), lambda i: (0, 0)),
            pl.BlockSpec((1, 4 * D), lambda i: (0, 0)),
        ],
        out_specs=[
            pl.BlockSpec((RB, D), lambda i: (i, 0)),
            pl.BlockSpec((RB, D), lambda i: (i, 0)),
        ],
        out_shape=[
            jax.ShapeDtypeStruct((N, D), jnp.float32),
            jax.ShapeDtypeStruct((N, D), jnp.float32),
        ],
    )


def _make_scale0(N, D, RB):
    def body(degp_ref, h1_ref, u1_ref, disb_ref):
        deg = jnp.sum(degp_ref[0], axis=0)
        deg_safe = jnp.where(deg > 0, deg, 1.0)
        dis = jnp.where(deg > 0, lax.rsqrt(deg_safe), 0.0)
        disb = jnp.broadcast_to(dis[:, None], (RB, D))
        disb_ref[...] = disb
        u1_ref[...] = disb * h1_ref[...]

    return pl.pallas_call(
        body,
        grid=(N // RB,),
        in_specs=[
            pl.BlockSpec((1, NW, RB), lambda i: (i, 0, 0)),
            pl.BlockSpec((RB, D), lambda i: (i, 0)),
        ],
        out_specs=[
            pl.BlockSpec((RB, D), lambda i: (i, 0)),
            pl.BlockSpec((RB, D), lambda i: (i, 0)),
        ],
        out_shape=[
            jax.ShapeDtypeStruct((N, D), jnp.float32),
            jax.ShapeDtypeStruct((N, D), jnp.float32),
        ],
    )


def _make_mid(N, D, RB):
    def body(y_ref, disb_ref, tx1_ref, u2_ref):
        ysum = y_ref[0] + y_ref[1]
        disb = disb_ref[...]
        tx1 = -disb * ysum
        tx1_ref[...] = tx1
        u2_ref[...] = disb * tx1

    return pl.pallas_call(
        body,
        grid=(N // RB,),
        in_specs=[
            pl.BlockSpec((NC, RB, D), lambda i: (0, i, 0)),
            pl.BlockSpec((RB, D), lambda i: (i, 0)),
        ],
        out_specs=[
            pl.BlockSpec((RB, D), lambda i: (i, 0)),
            pl.BlockSpec((RB, D), lambda i: (i, 0)),
        ],
        out_shape=[
            jax.ShapeDtypeStruct((N, D), jnp.float32),
            jax.ShapeDtypeStruct((N, D), jnp.float32),
        ],
    )


def _make_final(N, D, RB):
    def body(x_ref, h1_ref, c1_ref, tx1_ref, disb_ref, y_ref, w_ref, b_ref,
             out_ref):
        h1 = h1_ref[...]
        tx2 = -2.0 * disb_ref[...] * (y_ref[0] + y_ref[1]) - h1
        a = jnp.concatenate([x_ref[...], h1, tx1_ref[...], tx2], axis=1)
        z = jnp.dot(a, w_ref[...],
                    preferred_element_type=jnp.float32) + b_ref[...]
        zi, zf, zc, zo = _gates(z, D)
        c2 = zf * c1_ref[...] + zi * zc
        h2 = zo * jnp.tanh(c2)
        out_ref[...] = jnp.maximum(h2, 0.0)

    return pl.pallas_call(
        body,
        grid=(N // RB,),
        in_specs=[
            pl.BlockSpec((RB, D), lambda i: (i, 0)),
            pl.BlockSpec((RB, D), lambda i: (i, 0)),
            pl.BlockSpec((RB, D), lambda i: (i, 0)),
            pl.BlockSpec((RB, D), lambda i: (i, 0)),
            pl.BlockSpec((RB, D), lambda i: (i, 0)),
            pl.BlockSpec((NC, RB, D), lambda i: (0, i, 0)),
            pl.BlockSpec((4 * D, 4 * D), lambda i: (0, 0)),
            pl.BlockSpec((1, 4 * D), lambda i: (0, 0)),
        ],
        out_specs=pl.BlockSpec((RB, D), lambda i: (i, 0)),
        out_shape=jax.ShapeDtypeStruct((N, D), jnp.float32),
    )


def kernel(edge_index_list, node_feats_list, edge_feats_list, nodes_mask_list,
           W_i, W_f, W_c, W_o, b_i, b_f, b_c, b_o,
           conv_i_W, conv_i_b, conv_f_W, conv_f_b,
           conv_c_W, conv_c_b, conv_o_W, conv_o_b):
    del nodes_mask_list
    Tn, N, D = node_feats_list.shape
    E = edge_index_list.shape[2]
    assert Tn == 2

    X0 = node_feats_list[0]
    X1 = node_feats_list[1]
    src = edge_index_list[1, 0]
    dst = edge_index_list[1, 1]
    ew = edge_feats_list[1]

    Wx = jnp.concatenate([W_i, W_f, W_c, W_o], axis=1)
    Wk = [jnp.concatenate([conv_i_W[k], conv_f_W[k],
                           conv_c_W[k], conv_o_W[k]], axis=1)
          for k in range(3)]
    Wbig = jnp.concatenate([Wx, Wk[0], Wk[1], Wk[2]], axis=0)
    bias = (jnp.concatenate([b_i, b_f, b_c, b_o], axis=1)
            + jnp.concatenate([conv_i_b, conv_f_b,
                               conv_c_b, conv_o_b])[None, :])

    RB = 1000
    deg_p = _make_deg(E, N, RB)(src, ew)
    H1, C1 = _make_dense0(N, D, RB)(X0, Wx, bias)
    U1, disb = _make_scale0(N, D, RB)(deg_p, H1)
    Y1 = _make_prop(E, N, D)(U1, src, dst, ew)
    Tx1, U2 = _make_mid(N, D, RB)(Y1, disb)
    Y2 = _make_prop(E, N, D)(U2, src, dst, ew)
    out = _make_final(N, D, RB)(X1, H1, C1, Tx1, disb, Y2, Wbig, bias)
    return out


# speedup vs baseline: 40.1553x; 2.6989x over previous
import functools

import jax
import jax.numpy as jnp
from jax import lax
from jax.experimental import pallas as pl
from jax.experimental.pallas import tpu as pltpu
from jax.experimental.pallas import tpu_sc as plsc

NC = 2
NS = 16
NW = NC * NS


def _sc_mesh():
    return plsc.VectorSubcoreMesh(
        core_axis_name="c", subcore_axis_name="s",
        num_cores=NC, num_subcores=NS)


def _make_deg(E, N, RB):
    EPT = E // NW
    assert EPT % 16 == 0 and N % 16 == 0 and N % RB == 0

    @functools.partial(
        pl.kernel, mesh=_sc_mesh(),
        out_type=jax.ShapeDtypeStruct((N // RB, NW, RB), jnp.float32),
        compiler_params=pltpu.CompilerParams(
            needs_layout_passes=False, use_tc_tiling_on_sc=False),
        scratch_types=[
            pltpu.VMEM((N,), jnp.float32),
            pltpu.VMEM((EPT,), jnp.int32),
            pltpu.VMEM((EPT,), jnp.float32),
        ],
    )
    def deg_kernel(src_hbm, ew_hbm, out_hbm, acc_v, src_all, ew_all):
        cid = lax.axis_index("c")
        sid = lax.axis_index("s")
        wid = sid * NC + cid
        base = wid * EPT
        pltpu.sync_copy(src_hbm.at[pl.ds(base, EPT)], src_all)
        pltpu.sync_copy(ew_hbm.at[pl.ds(base, EPT)], ew_all)
        zero = jnp.zeros((16,), jnp.float32)

        def zbody(i, carry):
            acc_v[pl.ds(i * 16, 16)] = zero
            return carry
        lax.fori_loop(0, N // 16, zbody, 0)

        def grp(g, c2):
            idx = src_all[pl.ds(g * 16, 16)]
            w = ew_all[pl.ds(g * 16, 16)]
            plsc.addupdate_scatter(acc_v, [idx], w)
            return c2
        lax.fori_loop(0, EPT // 16, grp, 0)

        for q in range(N // RB):
            pltpu.sync_copy(acc_v.at[pl.ds(q * RB, RB)], out_hbm.at[q, wid])

    return deg_kernel


def _make_prop(E, N, D):
    EPT = E // NW
    B = 40
    NB = EPT // B
    U = 5
    S = 3
    G = 2
    RPS = N // NS
    ZQ, ZREM = divmod(RPS, B)
    assert EPT % B == 0 and B % 8 == 0 and NB % U == 0 and D % 16 == 0

    @functools.partial(
        pl.kernel, mesh=_sc_mesh(),
        out_type=jax.ShapeDtypeStruct((NC, N, D), jnp.float32),
        compiler_params=pltpu.CompilerParams(
            needs_layout_passes=False, use_tc_tiling_on_sc=False),
        scratch_types=[
            pltpu.VMEM_SHARED((N, D), jnp.float32),
            pltpu.VMEM((U, B, D), jnp.float32),
            pltpu.VMEM((U, B), jnp.int32),
            pltpu.VMEM((U, B), jnp.int32),
            pltpu.VMEM((U, B), jnp.float32),
            pltpu.SemaphoreType.DMA((U,)),
            pltpu.SemaphoreType.DMA((U,)),
            pltpu.SemaphoreType.DMA((U,)),
            pltpu.SemaphoreType.DMA((U,)),
            pltpu.SemaphoreType.DMA((U,)),
        ],
    )
    def prop_kernel(u_hbm, src_hbm, dst_hbm, ew_hbm, out_hbm,
                    acc_sh, rows, sidx, didx, ewr,
                    gsem, xsem, dsem, wsem, ssem):
        cid = lax.axis_index("c")
        sid = lax.axis_index("s")
        wid = sid * NC + cid
        base = wid * EPT

        zero = jnp.zeros((16,), jnp.float32)

        def zrow(i, carry):
            for c in range(D // 16):
                rows[0, i, pl.ds(c * 16, 16)] = zero
            return carry
        lax.fori_loop(0, B, zrow, 0)
        for q in range(ZQ):
            pltpu.sync_copy(rows.at[0],
                            acc_sh.at[pl.ds(sid * RPS + q * B, B)])
        if ZREM:
            pltpu.sync_copy(rows.at[0, pl.ds(0, ZREM)],
                            acc_sh.at[pl.ds(sid * RPS + ZQ * B, ZREM)])
        plsc.subcore_barrier()

        def issue_small(blk, slot):
            pltpu.async_copy(src_hbm.at[pl.ds(base + blk * B, B)],
                             sidx.at[slot], xsem.at[slot])
            pltpu.async_copy(dst_hbm.at[pl.ds(base + blk * B, B)],
                             didx.at[slot], dsem.at[slot])
            pltpu.async_copy(ew_hbm.at[pl.ds(base + blk * B, B)],
                             ewr.at[slot], wsem.at[slot])

        def issue_gather(blk, slot):
            pltpu.make_async_copy(src_hbm.at[pl.ds(base + blk * B, B)],
                                  sidx.at[slot], xsem.at[slot]).wait()
            pltpu.async_copy(u_hbm.at[sidx.at[slot]],
                             rows.at[slot], gsem.at[slot])

        def wait_gather(blk, slot):
            pltpu.make_async_copy(u_hbm.at[sidx.at[slot]],
                                  rows.at[slot], gsem.at[slot]).wait()
            pltpu.make_async_copy(ew_hbm.at[pl.ds(base + blk * B, B)],
                                  ewr.at[slot], wsem.at[slot]).wait()

        def wait_scatter(slot):
            pltpu.make_async_copy(rows.at[slot], acc_sh.at[didx.at[slot]],
                                  ssem.at[slot]).wait()

        for a in range(S):
            issue_small(a, a)
        for a in range(G):
            issue_gather(a, a)

        def scale_16(a, w16, row0):
            for j in range(16):
                w = w16[j]
                row = row0 + j
                for c in range(D // 16):
                    rows[a, row, pl.ds(c * 16, 16)] = (
                        rows[a, row, pl.ds(c * 16, 16)] * w)

        def outer(ob, carry):
            for a in range(U):
                b = ob * U + a
                slot_s = (a + S) % U
                slot_g = (a + G) % U

                @pl.when((b >= U - S) & (b + S < NB))
                def _():
                    wait_scatter(slot_s)

                @pl.when(b + S < NB)
                def _():
                    issue_small(b + S, slot_s)

                @pl.when(b + G < NB)
                def _():
                    issue_gather(b + G, slot_g)

                wait_gather(b, a)

                def scale(g, c2):
                    scale_16(a, ewr[a, pl.ds(g * 16, 16)], g * 16)
                    return c2
                lax.fori_loop(0, B // 16, scale, 0)
                if B % 16:
                    rem = B % 16
                    w16 = ewr[a, pl.ds(B - 16, 16)]
                    for j in range(16 - rem, 16):
                        w = w16[j]
                        row = B - 16 + j
                        for c in range(D // 16):
                            rows[a, row, pl.ds(c * 16, 16)] = (
                                rows[a, row, pl.ds(c * 16, 16)] * w)

                pltpu.make_async_copy(dst_hbm.at[pl.ds(base + b * B, B)],
                                      didx.at[a], dsem.at[a]).wait()
                pltpu.async_copy(rows.at[a], acc_sh.at[didx.at[a]],
                                 ssem.at[a], add=True)
            return carry
        lax.fori_loop(0, NB // U, outer, 0)

        for x in range(NB - U, NB):
            wait_scatter(x % U)

        plsc.subcore_barrier()
        pltpu.sync_copy(acc_sh.at[pl.ds(sid * RPS, RPS)],
                        out_hbm.at[cid, pl.ds(sid * RPS, RPS)])

    return prop_kernel


def _gates(z, D):
    zi = jax.nn.sigmoid(z[:, :D])
    zf = jax.nn.sigmoid(z[:, D:2 * D])
    zc = jnp.tanh(z[:, 2 * D:3 * D])
    zo = jax.nn.sigmoid(z[:, 3 * D:])
    return zi, zf, zc, zo


def _make_dense0(N, D, RB):
    def body(x_ref, w_ref, b_ref, h1_ref, c1_ref):
        z = jnp.dot(x_ref[...], w_ref[...],
                    preferred_element_type=jnp.float32) + b_ref[...]
        zi, zf, zc, zo = _gates(z, D)
        c1 = zi * zc
        c1_ref[...] = c1
        h1_ref[...] = zo * jnp.tanh(c1)

    return pl.pallas_call(
        body,
        grid=(N // RB,),
        in_specs=[
            pl.BlockSpec((RB, D), lambda i: (i, 0)),
            pl.BlockSpec((D, 4 * D), lambda i: (0, 0)),
            pl.BlockSpec((1, 4 * D), lambda i: (0, 0)),
        ],
        out_specs=[
            pl.BlockSpec((RB, D), lambda i: (i, 0)),
            pl.BlockSpec((RB, D), lambda i: (i, 0)),
        ],
        out_shape=[
            jax.ShapeDtypeStruct((N, D), jnp.float32),
            jax.ShapeDtypeStruct((N, D), jnp.float32),
        ],
    )


def _make_scale0(N, D, RB):
    def body(degp_ref, h1_ref, u1_ref, disb_ref):
        deg = jnp.sum(degp_ref[0], axis=0)
        deg_safe = jnp.where(deg > 0, deg, 1.0)
        dis = jnp.where(deg > 0, lax.rsqrt(deg_safe), 0.0)
        disb = jnp.broadcast_to(dis[:, None], (RB, D))
        disb_ref[...] = disb
        u1_ref[...] = disb * h1_ref[...]

    return pl.pallas_call(
        body,
        grid=(N // RB,),
        in_specs=[
            pl.BlockSpec((1, NW, RB), lambda i: (i, 0, 0)),
            pl.BlockSpec((RB, D), lambda i: (i, 0)),
        ],
        out_specs=[
            pl.BlockSpec((RB, D), lambda i: (i, 0)),
            pl.BlockSpec((RB, D), lambda i: (i, 0)),
        ],
        out_shape=[
            jax.ShapeDtypeStruct((N, D), jnp.float32),
            jax.ShapeDtypeStruct((N, D), jnp.float32),
        ],
    )


def _make_mid(N, D, RB):
    def body(y_ref, disb_ref, tx1_ref, u2_ref):
        ysum = y_ref[0] + y_ref[1]
        disb = disb_ref[...]
        tx1 = -disb * ysum
        tx1_ref[...] = tx1
        u2_ref[...] = disb * tx1

    return pl.pallas_call(
        body,
        grid=(N // RB,),
        in_specs=[
            pl.BlockSpec((NC, RB, D), lambda i: (0, i, 0)),
            pl.BlockSpec((RB, D), lambda i: (i, 0)),
        ],
        out_specs=[
            pl.BlockSpec((RB, D), lambda i: (i, 0)),
            pl.BlockSpec((RB, D), lambda i: (i, 0)),
        ],
        out_shape=[
            jax.ShapeDtypeStruct((N, D), jnp.float32),
            jax.ShapeDtypeStruct((N, D), jnp.float32),
        ],
    )


def _make_final(N, D, RB):
    def body(x_ref, h1_ref, c1_ref, tx1_ref, disb_ref, y_ref, w_ref, b_ref,
             out_ref):
        h1 = h1_ref[...]
        tx2 = -2.0 * disb_ref[...] * (y_ref[0] + y_ref[1]) - h1
        a = jnp.concatenate([x_ref[...], h1, tx1_ref[...], tx2], axis=1)
        z = jnp.dot(a, w_ref[...],
                    preferred_element_type=jnp.float32) + b_ref[...]
        zi, zf, zc, zo = _gates(z, D)
        c2 = zf * c1_ref[...] + zi * zc
        h2 = zo * jnp.tanh(c2)
        out_ref[...] = jnp.maximum(h2, 0.0)

    return pl.pallas_call(
        body,
        grid=(N // RB,),
        in_specs=[
            pl.BlockSpec((RB, D), lambda i: (i, 0)),
            pl.BlockSpec((RB, D), lambda i: (i, 0)),
            pl.BlockSpec((RB, D), lambda i: (i, 0)),
            pl.BlockSpec((RB, D), lambda i: (i, 0)),
            pl.BlockSpec((RB, D), lambda i: (i, 0)),
            pl.BlockSpec((NC, RB, D), lambda i: (0, i, 0)),
            pl.BlockSpec((4 * D, 4 * D), lambda i: (0, 0)),
            pl.BlockSpec((1, 4 * D), lambda i: (0, 0)),
        ],
        out_specs=pl.BlockSpec((RB, D), lambda i: (i, 0)),
        out_shape=jax.ShapeDtypeStruct((N, D), jnp.float32),
    )


def kernel(edge_index_list, node_feats_list, edge_feats_list, nodes_mask_list,
           W_i, W_f, W_c, W_o, b_i, b_f, b_c, b_o,
           conv_i_W, conv_i_b, conv_f_W, conv_f_b,
           conv_c_W, conv_c_b, conv_o_W, conv_o_b):
    del nodes_mask_list
    Tn, N, D = node_feats_list.shape
    E = edge_index_list.shape[2]
    assert Tn == 2

    X0 = node_feats_list[0]
    X1 = node_feats_list[1]
    src = edge_index_list[1, 0]
    dst = edge_index_list[1, 1]
    ew = edge_feats_list[1]

    Wx = jnp.concatenate([W_i, W_f, W_c, W_o], axis=1)
    Wk = [jnp.concatenate([conv_i_W[k], conv_f_W[k],
                           conv_c_W[k], conv_o_W[k]], axis=1)
          for k in range(3)]
    Wbig = jnp.concatenate([Wx, Wk[0], Wk[1], Wk[2]], axis=0)
    bias = (jnp.concatenate([b_i, b_f, b_c, b_o], axis=1)
            + jnp.concatenate([conv_i_b, conv_f_b,
                               conv_c_b, conv_o_b])[None, :])

    RB = 1000
    deg_p = _make_deg(E, N, RB)(src, ew)
    H1, C1 = _make_dense0(N, D, RB)(X0, Wx, bias)
    U1, disb = _make_scale0(N, D, RB)(deg_p, H1)
    Y1 = _make_prop(E, N, D)(U1, src, dst, ew)
    Tx1, U2 = _make_mid(N, D, RB)(Y1, disb)
    Y2 = _make_prop(E, N, D)(U2, src, dst, ew)
    out = _make_final(N, D, RB)(X1, H1, C1, Tx1, disb, Y2, Wbig, bias)
    return out
